# Initial kernel scaffold; baseline (speedup 1.0000x reference)
import jax
import jax.numpy as jnp
from jax.experimental import pallas as pl


def kernel(x, edge_index, W, att_src, att_dst, bias, gamma, beta):
    raise NotImplementedError("write your pallas kernel here")


# speedup vs baseline: 23.8910x; 23.8910x over previous
import jax
import jax.numpy as jnp
from jax import lax
from jax.experimental import pallas as pl
from jax.experimental.pallas import tpu as pltpu
from jax.experimental.pallas import tpu_sc as plsc

N = 10000
NP = 10240
E = 320000
HEADS = 4
HID = 128
NW = 32
EPW = E // NW
CHUNK = 80
NCHUNK = EPW // CHUNK

_SC_PARAMS = pltpu.CompilerParams(needs_layout_passes=False)


def _mesh():
    return plsc.VectorSubcoreMesh(core_axis_name="c", subcore_axis_name="s")


def _tc1_body(x_ref, w_ref, as_ref, ad_ref, ht_ref, asrc_ref, adst_ref, c_ref):
    xb = x_ref[...]
    for h in range(HEADS):
        hh = jnp.dot(xb, w_ref[:, h, :], preferred_element_type=jnp.float32)
        ht_ref[h] = hh
        s = jnp.sum(hh * as_ref[h][None, :], axis=1)
        d = jnp.sum(hh * ad_ref[h][None, :], axis=1)
        asrc_ref[h, :] = s
        adst_ref[h, :] = d
        al = s + d
        c_ref[h, :] = jnp.where(al >= 0, al, 0.2 * al)


def _tc1(xp, w3, att_s, att_d):
    bn = 1024
    return pl.pallas_call(
        _tc1_body,
        grid=(NP // bn,),
        in_specs=[
            pl.BlockSpec((bn, HID), lambda i: (i, 0)),
            pl.BlockSpec((HID, HEADS, HID), lambda i: (0, 0, 0)),
            pl.BlockSpec((HEADS, HID), lambda i: (0, 0)),
            pl.BlockSpec((HEADS, HID), lambda i: (0, 0)),
        ],
        out_specs=[
            pl.BlockSpec((HEADS, bn, HID), lambda i: (0, i, 0)),
            pl.BlockSpec((HEADS, bn), lambda i: (0, i)),
            pl.BlockSpec((HEADS, bn), lambda i: (0, i)),
            pl.BlockSpec((HEADS, bn), lambda i: (0, i)),
        ],
        out_shape=[
            jax.ShapeDtypeStruct((HEADS, NP, HID), jnp.float32),
            jax.ShapeDtypeStruct((HEADS, NP), jnp.float32),
            jax.ShapeDtypeStruct((HEADS, NP), jnp.float32),
            jax.ShapeDtypeStruct((HEADS, NP), jnp.float32),
        ],
    )(xp, w3, att_s, att_d)


def _sc1_body(src_hbm, dst_hbm, asrc_hbm, adst_hbm, c_hbm,
              ex_hbm, denp_hbm,
              src_v, dst_v, as_v, ad_v, c_v, den_v, ex_v):
    cid = lax.axis_index("c")
    sid = lax.axis_index("s")
    wid = sid * 2 + cid
    pltpu.sync_copy(src_hbm.at[wid], src_v)
    pltpu.sync_copy(dst_hbm.at[wid], dst_v)
    zero16 = jnp.zeros((16,), jnp.float32)

    def zbody(g, _):
        for h in range(HEADS):
            den_v[h, pl.ds(g * 16, 16)] = zero16
        return 0
    lax.fori_loop(0, N // 16, zbody, 0)

    for h in range(HEADS):
        pltpu.sync_copy(asrc_hbm.at[pl.ds(h * NP, NP)], as_v)
        pltpu.sync_copy(adst_hbm.at[pl.ds(h * NP, NP)], ad_v)
        pltpu.sync_copy(c_hbm.at[pl.ds(h * NP, NP)], c_v)
        hrow = jnp.full((16,), h, jnp.int32)

        def ebody(g, _):
            sl = pl.ds(g * 16, 16)
            s16 = src_v[0, sl]
            d16 = dst_v[0, sl]
            a1 = plsc.load_gather(as_v, [s16])
            a2 = plsc.load_gather(ad_v, [d16])
            cc = plsc.load_gather(c_v, [d16])
            al = a1 + a2
            al = jnp.where(al >= 0, al, 0.2 * al)
            e = jnp.exp(al - cc)
            ex_v[0, sl] = e
            plsc.addupdate_scatter(den_v, [hrow, d16], e)
            return 0
        lax.fori_loop(0, EPW // 16, ebody, 0)

        pltpu.sync_copy(ex_v, ex_hbm.at[wid, h])
    pltpu.sync_copy(den_v, denp_hbm.at[wid])


def _sc1(src3, dst3, asrcf, adstf, cf):
    f = pl.kernel(
        _sc1_body,
        out_type=[
            jax.ShapeDtypeStruct((NW, HEADS, 1, EPW), jnp.float32),
            jax.ShapeDtypeStruct((NW, HEADS, N), jnp.float32),
        ],
        mesh=_mesh(),
        compiler_params=_SC_PARAMS,
        scratch_types=[
            pltpu.VMEM((1, EPW), jnp.int32),
            pltpu.VMEM((1, EPW), jnp.int32),
            pltpu.VMEM((NP,), jnp.float32),
            pltpu.VMEM((NP,), jnp.float32),
            pltpu.VMEM((NP,), jnp.float32),
            pltpu.VMEM((HEADS, N), jnp.float32),
            pltpu.VMEM((1, EPW), jnp.float32),
        ],
    )
    return f(src3, dst3, asrcf, adstf, cf)


def _tc2_body(denp_ref, inv_ref):
    dsum = jnp.sum(denp_ref[...], axis=0) + 1.0
    inv = 1.0 / dsum
    inv_ref[...] = jnp.concatenate(
        [inv, jnp.zeros((HEADS, NP - N), jnp.float32)], axis=1)


def _tc2(denp):
    return pl.pallas_call(
        _tc2_body,
        grid=(1,),
        in_specs=[pl.BlockSpec((NW, HEADS, N), lambda i: (0, 0, 0))],
        out_specs=pl.BlockSpec((HEADS, NP), lambda i: (0, 0)),
        out_shape=jax.ShapeDtypeStruct((HEADS, NP), jnp.float32),
    )(denp)


def _sc1b_body(dst_hbm, inv_hbm, ex_hbm, coef_hbm,
               dst_v, inv_v, ex_v, co_v):
    cid = lax.axis_index("c")
    sid = lax.axis_index("s")
    wid = sid * 2 + cid
    pltpu.sync_copy(dst_hbm.at[wid], dst_v)
    pltpu.sync_copy(inv_hbm, inv_v)
    for h in range(HEADS):
        pltpu.sync_copy(ex_hbm.at[wid, h], ex_v)
        hoff = h * NP

        def gbody(g, _):
            sl = pl.ds(g * 16, 16)
            d16 = dst_v[0, sl]
            iv = plsc.load_gather(inv_v, [d16 + hoff])
            co_v[0, sl] = ex_v[0, sl] * iv * 0.25
            return 0
        lax.fori_loop(0, EPW // 16, gbody, 0)
        pltpu.sync_copy(co_v, coef_hbm.at[wid, h])


def _sc1b(dst3, invf, ex):
    f = pl.kernel(
        _sc1b_body,
        out_type=jax.ShapeDtypeStruct((NW, HEADS, 1, EPW), jnp.float32),
        mesh=_mesh(),
        compiler_params=_SC_PARAMS,
        scratch_types=[
            pltpu.VMEM((1, EPW), jnp.int32),
            pltpu.VMEM((HEADS * NP,), jnp.float32),
            pltpu.VMEM((1, EPW), jnp.float32),
            pltpu.VMEM((1, EPW), jnp.float32),
        ],
    )
    return f(dst3, invf, ex)


def _sc2_body(src_hbm, dst_hbm, ht2_hbm, coef_hbm,
              part_hbm,
              src_v, dst_v, co_v, idx_v, didx_v, rows_v, acc_sh, sem):
    cid = lax.axis_index("c")
    sid = lax.axis_index("s")
    wid = sid * 2 + cid

    zero16 = jnp.zeros((16,), jnp.float32)

    def zbody(r, _):
        for cc in range(HID // 16):
            rows_v[r, pl.ds(cc * 16, 16)] = zero16
        return 0
    lax.fori_loop(0, CHUNK, zbody, 0)
    stripe = NP // 16
    for b in range(stripe // CHUNK):
        pltpu.sync_copy(rows_v, acc_sh.at[pl.ds(sid * stripe + b * CHUNK, CHUNK)])
    plsc.subcore_barrier()

    pltpu.sync_copy(src_hbm.at[wid], src_v)
    pltpu.sync_copy(dst_hbm.at[wid], dst_v)

    for h in range(HEADS):
        pltpu.sync_copy(coef_hbm.at[wid, h], co_v)
        hoff = h * NP

        def cbody(i, _):
            for j in range(CHUNK // 16):
                sl16 = pl.ds(j * 16, 16)
                esl = pl.ds(i * CHUNK + j * 16, 16)
                s16 = src_v[0, esl]
                idx_v[sl16] = s16 + hoff
                didx_v[sl16] = dst_v[0, esl]
            pltpu.async_copy(ht2_hbm.at[idx_v], rows_v, sem).wait()
            for j in range(CHUNK // 16):
                cov = co_v[0, pl.ds(i * CHUNK + j * 16, 16)]
                for k in range(16):
                    r = j * 16 + k
                    bco = jnp.broadcast_to(cov[k], (16,))
                    for cc in range(HID // 16):
                        slc = pl.ds(cc * 16, 16)
                        rows_v[r, slc] = rows_v[r, slc] * bco
            pltpu.sync_copy(rows_v, acc_sh.at[didx_v], add=True)
            return 0
        lax.fori_loop(0, NCHUNK, cbody, 0)

    plsc.subcore_barrier()
    stripe_sl = pl.ds(sid * stripe, stripe)
    pltpu.sync_copy(acc_sh.at[stripe_sl], part_hbm.at[cid, stripe_sl])


def _sc2(src3, dst3, ht2, coef):
    f = pl.kernel(
        _sc2_body,
        out_type=jax.ShapeDtypeStruct((2, NP, HID), jnp.float32),
        mesh=_mesh(),
        compiler_params=_SC_PARAMS,
        scratch_types=[
            pltpu.VMEM((1, EPW), jnp.int32),
            pltpu.VMEM((1, EPW), jnp.int32),
            pltpu.VMEM((1, EPW), jnp.float32),
            pltpu.VMEM((CHUNK,), jnp.int32),
            pltpu.VMEM((CHUNK,), jnp.int32),
            pltpu.VMEM((CHUNK, HID), jnp.float32),
            pltpu.VMEM_SHARED((NP, HID), jnp.float32),
            pltpu.SemaphoreType.DMA,
        ],
    )
    return f(src3, dst3, ht2, coef)


def _tc3_body(part_ref, ht_ref, inv_ref, b_ref, g_ref, be_ref, out_ref):
    i = pl.program_id(0)
    acc = part_ref[0] + part_ref[1]
    for h in range(HEADS):
        invb = inv_ref[h, pl.ds(i * 1024, 1024)]
        acc = acc + 0.25 * invb[:, None] * ht_ref[h]
    acc = acc + b_ref[...][None, :]
    acc = jnp.where(acc > 0, acc, jnp.exp(jnp.minimum(acc, 0.0)) - 1.0)
    mu = jnp.mean(acc, axis=1, keepdims=True)
    var = jnp.mean((acc - mu) ** 2, axis=1, keepdims=True)
    out_ref[...] = ((acc - mu) * lax.rsqrt(var + 1e-5) * g_ref[...][None, :]
                    + be_ref[...][None, :])


def _tc3(part, ht, inv, bias, gamma, beta):
    bn = 1024
    return pl.pallas_call(
        _tc3_body,
        grid=(NP // bn,),
        in_specs=[
            pl.BlockSpec((2, bn, HID), lambda i: (0, i, 0)),
            pl.BlockSpec((HEADS, bn, HID), lambda i: (0, i, 0)),
            pl.BlockSpec((HEADS, NP), lambda i: (0, 0)),
            pl.BlockSpec((HID,), lambda i: (0,)),
            pl.BlockSpec((HID,), lambda i: (0,)),
            pl.BlockSpec((HID,), lambda i: (0,)),
        ],
        out_specs=pl.BlockSpec((bn, HID), lambda i: (i, 0)),
        out_shape=jax.ShapeDtypeStruct((NP, HID), jnp.float32),
    )(part, ht, inv, bias, gamma, beta)


def kernel(x, edge_index, W, att_src, att_dst, bias, gamma, beta):
    xp = jnp.pad(x, ((0, NP - N), (0, 0)))
    w3 = W.reshape(HID, HEADS, HID)
    att_s = att_src.reshape(HEADS, HID)
    att_d = att_dst.reshape(HEADS, HID)
    src3 = edge_index[0].reshape(NW, 1, EPW)
    dst3 = edge_index[1].reshape(NW, 1, EPW)

    ht, asrc, adst, c = _tc1(xp, w3, att_s, att_d)
    ex, denp = _sc1(src3, dst3, asrc.reshape(-1), adst.reshape(-1),
                    c.reshape(-1))
    inv = _tc2(denp)
    coef = _sc1b(dst3, inv.reshape(-1), ex)
    ht2 = ht.reshape(HEADS * NP, HID)
    part = _sc2(src3, dst3, ht2, coef)
    out = _tc3(part, ht, inv, bias, gamma, beta)
    return out[:N]
